# baseline (speedup 1.0000x reference)
import jax
import jax.numpy as jnp
import numpy as np
from jax import lax
from jax.experimental import pallas as pl
from jax.experimental.pallas import tpu as pltpu

C_IN, C_OUT, KH, KW = 3, 6, 5, 5
H, W = 32, 32
HO, WO = H - KH + 1, W - KW + 1
HW = H * W
N_CLS = 3
KWIN = KH * W
NF = 8 * W
FT = HO * NF
ONE_COL = C_OUT * W
NL = 8
TB = 256

_WIN = np.arange(W)[:, None]
_WO = np.arange(W)[None, :]
_M = np.stack([((_WIN - _WO) == j) & (_WO < WO) for j in range(KW)])
_MASK = _M.reshape(KW, W * W).astype(np.float32)


def _fused_body(x_ref, wc_ref, brow_ref, w2_ref, o_ref, feat_ref):
    for h in range(HO):
        acc = brow_ref[0:1, :]
        for c in range(C_IN):
            acc = acc + lax.dot_general(
                x_ref[:, c * HW + h * W:c * HW + h * W + KWIN], wc_ref[c],
                (((1,), (0,)), ((), ())), preferred_element_type=jnp.float32)
        feat_ref[:, h * NF:(h + 1) * NF] = jnp.maximum(acc, 0.0)

    logits = lax.dot_general(
        feat_ref[...], w2_ref[...],
        (((1,), (0,)), ((), ())), preferred_element_type=jnp.float32)
    lg = logits[:, :N_CLS]
    s = lg - jnp.max(lg, axis=-1, keepdims=True)
    o_ref[...] = s - jnp.log(jnp.sum(jnp.exp(s), axis=-1, keepdims=True))


def _build_conv_weights(w1):
    t = jnp.einsum('ocdj,jp->cdpo', w1.astype(jnp.float32), _MASK)
    t = t.reshape(C_IN, KH, W, W, C_OUT)
    t = jnp.swapaxes(t, 3, 4)
    t = t.reshape(C_IN, KWIN, C_OUT * W)
    return jnp.pad(t, ((0, 0), (0, 0), (0, NF - C_OUT * W)))


def _build_bias_row(b1):
    brow = jnp.where(jnp.tile(jnp.arange(W) < WO, C_OUT),
                     jnp.repeat(b1.astype(jnp.float32), W), 0.0)
    brow = jnp.pad(brow, (0, NF - C_OUT * W)).at[ONE_COL].set(1.0)
    return jnp.broadcast_to(brow[None, :], (8, NF))


def _build_linear_weights(w2, b2):
    w2r = w2.astype(jnp.float32).reshape(N_CLS, C_OUT, HO, WO)
    w2t = jnp.transpose(w2r, (2, 1, 3, 0))
    w2t = jnp.pad(w2t, ((0, 0), (0, 2), (0, W - WO), (0, NL - N_CLS)))
    w2f = w2t.reshape(FT, NL)
    return w2f.at[ONE_COL, :N_CLS].set(b2.astype(jnp.float32))


@jax.jit
def _forward(x, w1, b1, w2, b2):
    n = x.shape[0]
    tb = min(TB, ((n + 7) // 8) * 8)
    n_pad = (-n) % tb
    n_tiles = (n + n_pad) // tb

    x2 = x.reshape(n, C_IN * HW)
    if n_pad:
        x2 = jnp.pad(x2, ((0, n_pad), (0, 0)))

    wc = _build_conv_weights(w1)
    brow = _build_bias_row(b1)
    w2f = _build_linear_weights(w2, b2)

    out = pl.pallas_call(
        _fused_body,
        out_shape=jax.ShapeDtypeStruct((n + n_pad, N_CLS), jnp.float32),
        grid=(n_tiles,),
        in_specs=[
            pl.BlockSpec((tb, C_IN * HW), lambda b: (b, 0)),
            pl.BlockSpec((C_IN, KWIN, NF), lambda b: (0, 0, 0)),
            pl.BlockSpec((8, NF), lambda b: (0, 0)),
            pl.BlockSpec((FT, NL), lambda b: (0, 0)),
        ],
        out_specs=pl.BlockSpec((tb, N_CLS), lambda b: (b, 0)),
        scratch_shapes=[
            pltpu.VMEM((tb, FT), jnp.float32),
        ],
        compiler_params=pltpu.CompilerParams(
            dimension_semantics=("parallel",)),
    )(x2, wc, brow, w2f)
    return out[:n]


def kernel(x, w1, b1, w2, b2):
    return _forward(x, w1, b1, w2, b2)
